# baseline (speedup 1.0000x reference)
import functools

import jax
import jax.numpy as jnp
from jax import lax
from jax.experimental import pallas as pl
from jax.experimental.pallas import tpu as pltpu
from jax.experimental.pallas import tpu_sc as plsc

N_USERS = 100000
N_ITEMS = 1000000
EMB_DIM = 32
BATCH = 16384

_NC = 2
_NS = 16
_HD = EMB_DIM // _NC
_LPW = BATCH // _NS
_CHK = 256

_W = 1024
_WP = _W // 8
_NWIN_I = N_ITEMS // _W
_TAIL_I = N_ITEMS - _NWIN_I * _W
_NWIN_U = N_USERS // _W
_TAIL_U = N_USERS - _NWIN_U * _W
_JW_I = -(-_NWIN_I // _NS)
_JW_U = -(-_NWIN_U // _NS)
_SCR_I = _NWIN_I * _WP + _WP
_SCR_U = _NWIN_U * _WP + _WP


def _repack(win, pack, np_rows):
    lane = lax.iota(jnp.int32, 16)

    def rows(p2, carry):
        p = p2 * 2
        vals = []
        for qg in range(16):
            qd = (lane + qg) & 15
            vals.append(plsc.load_gather(win, [lane, qd + 8 * p]))
        for qg in range(16):
            qd = (lane + qg) & 15
            prow = (qd >> 3) + p
            scol = ((qd & 7) << 4) + lane
            plsc.store_scatter(pack, [prow, scol], vals[qg])
        return carry

    lax.fori_loop(0, np_rows // 2, rows, 0)


def _phase1(tab, scr, c, s, nwin, jw, win, pack, gsem, wsem):
    dbase = pl.multiple_of(c * _HD, _HD)

    def win_copy(j, buf, sem):
        w = s + _NS * j
        cb = pl.multiple_of(w * _W, _W)
        return pltpu.make_async_copy(
            tab.at[pl.ds(dbase, _HD), pl.ds(cb, _W)], buf, sem)

    def wb_copy(j, buf, sem):
        w = s + _NS * j
        rb = pl.multiple_of(w * _WP, _WP)
        return pltpu.make_async_copy(
            buf, scr.at[c, pl.ds(rb, _WP), :], sem)

    def start(j, buf, sem):
        @pl.when(s + _NS * j < nwin)
        def _():
            win_copy(j, buf, sem).start()

    def wait_g(j, buf, sem):
        @pl.when(s + _NS * j < nwin)
        def _():
            win_copy(j, buf, sem).wait()

    def start_wb(j, buf, sem):
        @pl.when(s + _NS * j < nwin)
        def _():
            wb_copy(j, buf, sem).start()

    def wait_wb(j, buf, sem):
        @pl.when(s + _NS * j < nwin)
        def _():
            wb_copy(j, buf, sem).wait()

    start(0, win[0], gsem[0])

    def body(p, carry):
        ja, jb = 2 * p, 2 * p + 1
        start(jb, win[1], gsem[1])

        @pl.when(p > 0)
        def _():
            wait_wb(ja - 2, pack[0], wsem[0])
        wait_g(ja, win[0], gsem[0])

        @pl.when(s + _NS * ja < nwin)
        def _():
            _repack(win[0], pack[0], _WP)
        start_wb(ja, pack[0], wsem[0])
        start(ja + 2, win[0], gsem[0])

        @pl.when(p > 0)
        def _():
            wait_wb(jb - 2, pack[1], wsem[1])
        wait_g(jb, win[1], gsem[1])

        @pl.when(s + _NS * jb < nwin)
        def _():
            _repack(win[1], pack[1], _WP)
        start_wb(jb, pack[1], wsem[1])
        return carry

    np2 = -(-jw // 2)
    lax.fori_loop(0, np2, body, 0)
    wait_wb(2 * np2 - 2, pack[0], wsem[0])
    wait_wb(2 * np2 - 1, pack[1], wsem[1])


def _tail(tail_ref, scr, c, s, row_base, win, pack, gsem, wsem):
    @pl.when(s == _NS - 1)
    def _():
        dbase = pl.multiple_of(c * _HD, _HD)
        cp = pltpu.make_async_copy(
            tail_ref.at[pl.ds(dbase, _HD), :], win, gsem)
        cp.start()
        cp.wait()
        _repack(win, pack, _WP)
        wb = pltpu.make_async_copy(
            pack, scr.at[c, pl.ds(row_base, _WP), :], wsem)
        wb.start()
        wb.wait()


def _phase2(idx, scr, out, c, s, fetch, blk, outb, gsem, wsem):
    for k in range(_LPW // _CHK):
        kb = k * _CHK
        for t in range(_CHK // 16):
            blk[pl.ds(t * 16, 16)] = idx[pl.ds(kb + t * 16, 16)] >> 3
        gc = pltpu.make_async_copy(scr.at[c].at[blk], fetch, gsem)
        gc.start()
        gc.wait()

        lane = lax.iota(jnp.int32, 16)

        def sel(jg, carry):
            jloc = lane + jg * 16
            iv = idx[pl.ds(kb + jg * 16, 16)]
            colb = (iv & 7) << 4
            orow = jloc >> 3
            ocolb = (jloc & 7) << 4
            vals = []
            for dg in range(_HD):
                dd = (lane + dg) & (_HD - 1)
                vals.append(plsc.load_gather(fetch, [jloc, colb + dd]))
            for dg in range(_HD):
                dd = (lane + dg) & (_HD - 1)
                plsc.store_scatter(outb, [orow, ocolb + dd], vals[dg])
            return carry

        lax.fori_loop(0, _CHK // 16, sel, 0)
        ob = pl.multiple_of((s * _LPW + k * _CHK) // 8, _CHK // 8)
        wb = pltpu.make_async_copy(
            outb, out.at[c, pl.ds(ob, _CHK // 8), :], wsem)
        wb.start()
        wb.wait()


def _kernel_body(uids, iids1, iids2, utab, itab, utail, itail,
                 uout, i1out, i2out, uscr, iscr,
                 idx_u, idx_1, idx_2, win0, win1, pack0, pack1,
                 fetch, blk, outb,
                 g0, g1, w0, w1, gf, wf):
    c = lax.axis_index("c")
    s = lax.axis_index("s")

    pltpu.sync_copy(uids.at[pl.ds(s * _LPW, _LPW)], idx_u)
    pltpu.sync_copy(iids1.at[pl.ds(s * _LPW, _LPW)], idx_1)
    pltpu.sync_copy(iids2.at[pl.ds(s * _LPW, _LPW)], idx_2)

    win = [win0, win1]
    pack = [pack0, pack1]
    gsem = [g0, g1]
    wsem = [w0, w1]

    _phase1(itab, iscr, c, s, _NWIN_I, _JW_I, win, pack, gsem, wsem)
    _phase1(utab, uscr, c, s, _NWIN_U, _JW_U, win, pack, gsem, wsem)
    _tail(itail, iscr, c, s, _NWIN_I * _WP, win0, pack0, g0, w0)
    _tail(utail, uscr, c, s, _NWIN_U * _WP, win0, pack0, g0, w0)

    plsc.subcore_barrier()

    _phase2(idx_u, uscr, uout, c, s, fetch, blk, outb, gf, wf)
    _phase2(idx_1, iscr, i1out, c, s, fetch, blk, outb, gf, wf)
    _phase2(idx_2, iscr, i2out, c, s, fetch, blk, outb, gf, wf)


@jax.jit
def _run(user_ids, item_ids_1, item_ids_2, user_emb, item_emb):
    mesh = plsc.VectorSubcoreMesh(core_axis_name="c", subcore_axis_name="s")
    f32 = jnp.float32
    i32 = jnp.int32
    utab = user_emb.T
    itab = item_emb.T
    utail = jnp.pad(utab[:, _NWIN_U * _W:], ((0, 0), (0, _W - _TAIL_U)))
    itail = jnp.pad(itab[:, _NWIN_I * _W:], ((0, 0), (0, _W - _TAIL_I)))
    call = functools.partial(
        pl.kernel,
        mesh=mesh,
        compiler_params=pltpu.CompilerParams(needs_layout_passes=False),
        out_type=(
            jax.ShapeDtypeStruct((_NC, BATCH // 8, 128), f32),
            jax.ShapeDtypeStruct((_NC, BATCH // 8, 128), f32),
            jax.ShapeDtypeStruct((_NC, BATCH // 8, 128), f32),
            jax.ShapeDtypeStruct((_NC, _SCR_U, 128), f32),
            jax.ShapeDtypeStruct((_NC, _SCR_I, 128), f32),
        ),
        scratch_types=[
            pltpu.VMEM((_LPW,), i32),
            pltpu.VMEM((_LPW,), i32),
            pltpu.VMEM((_LPW,), i32),
            pltpu.VMEM((_HD, _W), f32),
            pltpu.VMEM((_HD, _W), f32),
            pltpu.VMEM((_WP, 128), f32),
            pltpu.VMEM((_WP, 128), f32),
            pltpu.VMEM((_CHK, 128), f32),
            pltpu.VMEM((_CHK,), i32),
            pltpu.VMEM((_CHK // 8, 128), f32),
            pltpu.SemaphoreType.DMA,
            pltpu.SemaphoreType.DMA,
            pltpu.SemaphoreType.DMA,
            pltpu.SemaphoreType.DMA,
            pltpu.SemaphoreType.DMA,
            pltpu.SemaphoreType.DMA,
        ],
    )(_kernel_body)
    uo, i1o, i2o, _, _ = call(user_ids.astype(i32), item_ids_1, item_ids_2,
                              utab, itab, utail, itail)

    def assemble(o):
        h0 = o[0].reshape(BATCH, _HD)
        h1 = o[1].reshape(BATCH, _HD)
        return jnp.concatenate([h0, h1], axis=1)

    return (assemble(uo), assemble(i1o), assemble(i2o))


def kernel(user_ids, item_ids_1, item_ids_2, user_emb, item_emb):
    return _run(user_ids, item_ids_1, item_ids_2, user_emb, item_emb)


# speedup vs baseline: 1.2185x; 1.2185x over previous
import functools

import jax
import jax.numpy as jnp
from jax import lax
from jax.experimental import pallas as pl
from jax.experimental.pallas import tpu as pltpu
from jax.experimental.pallas import tpu_sc as plsc

N_USERS = 100000
N_ITEMS = 1000000
EMB_DIM = 32
BATCH = 16384

_NC = 2
_NS = 16
_HD = EMB_DIM // _NC
_LPW = BATCH // _NS
_CHK = 256

_W = 1024
_WP = _W // 8
_NWIN_I = N_ITEMS // _W
_TAIL_I = N_ITEMS - _NWIN_I * _W
_NWIN_U = N_USERS // _W
_TAIL_U = N_USERS - _NWIN_U * _W
_JW_I = -(-_NWIN_I // _NS)
_JW_U = -(-_NWIN_U // _NS)
_SCR_I = _NWIN_I * _WP + _WP
_SCR_U = _NWIN_U * _WP + _WP


def _repack(win, pack, np_rows):
    lane = lax.iota(jnp.int32, 16)

    def rows(p2, carry):
        p = p2 * 2
        vals = []
        for dpp in range(2):
            for qg in range(8):
                qd = (lane + qg) & 7
                vals.append(plsc.load_gather(win, [lane, qd + 8 * (p + dpp)]))
        i = 0
        for dpp in range(2):
            prow = jnp.full((16,), 0, jnp.int32) + (p + dpp)
            for qg in range(8):
                qd = (lane + qg) & 7
                plsc.store_scatter(pack, [prow, (qd << 4) + lane], vals[i])
                i += 1
        return carry

    lax.fori_loop(0, np_rows // 2, rows, 0)


def _phase1(tab, scr, c, s, nwin, jw, win, pack, gsem, wsem):
    dbase = pl.multiple_of(c * _HD, _HD)

    def win_copy(j, buf, sem):
        w = s + _NS * j
        cb = pl.multiple_of(w * _W, _W)
        return pltpu.make_async_copy(
            tab.at[pl.ds(dbase, _HD), pl.ds(cb, _W)], buf, sem)

    def wb_copy(j, buf, sem):
        w = s + _NS * j
        rb = pl.multiple_of(w * _WP, _WP)
        return pltpu.make_async_copy(
            buf, scr.at[c, pl.ds(rb, _WP), :], sem)

    def start(j, buf, sem):
        @pl.when(s + _NS * j < nwin)
        def _():
            win_copy(j, buf, sem).start()

    def wait_g(j, buf, sem):
        @pl.when(s + _NS * j < nwin)
        def _():
            win_copy(j, buf, sem).wait()

    def start_wb(j, buf, sem):
        @pl.when(s + _NS * j < nwin)
        def _():
            wb_copy(j, buf, sem).start()

    def wait_wb(j, buf, sem):
        @pl.when(s + _NS * j < nwin)
        def _():
            wb_copy(j, buf, sem).wait()

    start(0, win[0], gsem[0])

    def body(p, carry):
        ja, jb = 2 * p, 2 * p + 1
        start(jb, win[1], gsem[1])

        @pl.when(p > 0)
        def _():
            wait_wb(ja - 2, pack[0], wsem[0])
        wait_g(ja, win[0], gsem[0])

        @pl.when(s + _NS * ja < nwin)
        def _():
            _repack(win[0], pack[0], _WP)
        start_wb(ja, pack[0], wsem[0])
        start(ja + 2, win[0], gsem[0])

        @pl.when(p > 0)
        def _():
            wait_wb(jb - 2, pack[1], wsem[1])
        wait_g(jb, win[1], gsem[1])

        @pl.when(s + _NS * jb < nwin)
        def _():
            _repack(win[1], pack[1], _WP)
        start_wb(jb, pack[1], wsem[1])
        return carry

    np2 = -(-jw // 2)
    lax.fori_loop(0, np2, body, 0)
    wait_wb(2 * np2 - 2, pack[0], wsem[0])
    wait_wb(2 * np2 - 1, pack[1], wsem[1])


def _tail(tail_ref, scr, c, s, row_base, win, pack, gsem, wsem):
    @pl.when(s == _NS - 1)
    def _():
        dbase = pl.multiple_of(c * _HD, _HD)
        cp = pltpu.make_async_copy(
            tail_ref.at[pl.ds(dbase, _HD), :], win, gsem)
        cp.start()
        cp.wait()
        _repack(win, pack, _WP)
        wb = pltpu.make_async_copy(
            pack, scr.at[c, pl.ds(row_base, _WP), :], wsem)
        wb.start()
        wb.wait()


def _phase2(idx, scr, out, c, s, fetch, blk, outb, gsem, wsem):
    for k in range(_LPW // _CHK):
        kb = k * _CHK
        for t in range(_CHK // 16):
            blk[pl.ds(t * 16, 16)] = idx[pl.ds(kb + t * 16, 16)] >> 3
        gc = pltpu.make_async_copy(scr.at[c].at[blk], fetch, gsem)
        gc.start()
        gc.wait()

        lane = lax.iota(jnp.int32, 16)

        def sel(jg, carry):
            jloc = lane + jg * 16
            iv = idx[pl.ds(kb + jg * 16, 16)]
            colb = (iv & 7) << 4
            orow = jloc >> 3
            ocolb = (jloc & 7) << 4
            vals = []
            for dg in range(_HD):
                dd = (lane + dg) & (_HD - 1)
                vals.append(plsc.load_gather(fetch, [jloc, colb + dd]))
            for dg in range(_HD):
                dd = (lane + dg) & (_HD - 1)
                plsc.store_scatter(outb, [orow, ocolb + dd], vals[dg])
            return carry

        lax.fori_loop(0, _CHK // 16, sel, 0)
        ob = pl.multiple_of((s * _LPW + k * _CHK) // 8, _CHK // 8)
        wb = pltpu.make_async_copy(
            outb, out.at[c, pl.ds(ob, _CHK // 8), :], wsem)
        wb.start()
        wb.wait()


def _kernel_body(uids, iids1, iids2, utab, itab, utail, itail,
                 uout, i1out, i2out, uscr, iscr,
                 idx_u, idx_1, idx_2, win0, win1, pack0, pack1,
                 fetch, blk, outb,
                 g0, g1, w0, w1, gf, wf):
    c = lax.axis_index("c")
    s = lax.axis_index("s")

    pltpu.sync_copy(uids.at[pl.ds(s * _LPW, _LPW)], idx_u)
    pltpu.sync_copy(iids1.at[pl.ds(s * _LPW, _LPW)], idx_1)
    pltpu.sync_copy(iids2.at[pl.ds(s * _LPW, _LPW)], idx_2)

    win = [win0, win1]
    pack = [pack0, pack1]
    gsem = [g0, g1]
    wsem = [w0, w1]

    _phase1(itab, iscr, c, s, _NWIN_I, _JW_I, win, pack, gsem, wsem)
    _phase1(utab, uscr, c, s, _NWIN_U, _JW_U, win, pack, gsem, wsem)
    _tail(itail, iscr, c, s, _NWIN_I * _WP, win0, pack0, g0, w0)
    _tail(utail, uscr, c, s, _NWIN_U * _WP, win0, pack0, g0, w0)

    plsc.subcore_barrier()

    _phase2(idx_u, uscr, uout, c, s, fetch, blk, outb, gf, wf)
    _phase2(idx_1, iscr, i1out, c, s, fetch, blk, outb, gf, wf)
    _phase2(idx_2, iscr, i2out, c, s, fetch, blk, outb, gf, wf)


@jax.jit
def _run(user_ids, item_ids_1, item_ids_2, user_emb, item_emb):
    mesh = plsc.VectorSubcoreMesh(core_axis_name="c", subcore_axis_name="s")
    f32 = jnp.float32
    i32 = jnp.int32
    utab = user_emb.T
    itab = item_emb.T
    utail = jnp.pad(utab[:, _NWIN_U * _W:], ((0, 0), (0, _W - _TAIL_U)))
    itail = jnp.pad(itab[:, _NWIN_I * _W:], ((0, 0), (0, _W - _TAIL_I)))
    call = functools.partial(
        pl.kernel,
        mesh=mesh,
        compiler_params=pltpu.CompilerParams(needs_layout_passes=False),
        out_type=(
            jax.ShapeDtypeStruct((_NC, BATCH // 8, 128), f32),
            jax.ShapeDtypeStruct((_NC, BATCH // 8, 128), f32),
            jax.ShapeDtypeStruct((_NC, BATCH // 8, 128), f32),
            jax.ShapeDtypeStruct((_NC, _SCR_U, 128), f32),
            jax.ShapeDtypeStruct((_NC, _SCR_I, 128), f32),
        ),
        scratch_types=[
            pltpu.VMEM((_LPW,), i32),
            pltpu.VMEM((_LPW,), i32),
            pltpu.VMEM((_LPW,), i32),
            pltpu.VMEM((_HD, _W), f32),
            pltpu.VMEM((_HD, _W), f32),
            pltpu.VMEM((_WP, 128), f32),
            pltpu.VMEM((_WP, 128), f32),
            pltpu.VMEM((_CHK, 128), f32),
            pltpu.VMEM((_CHK,), i32),
            pltpu.VMEM((_CHK // 8, 128), f32),
            pltpu.SemaphoreType.DMA,
            pltpu.SemaphoreType.DMA,
            pltpu.SemaphoreType.DMA,
            pltpu.SemaphoreType.DMA,
            pltpu.SemaphoreType.DMA,
            pltpu.SemaphoreType.DMA,
        ],
    )(_kernel_body)
    uo, i1o, i2o, _, _ = call(user_ids.astype(i32), item_ids_1, item_ids_2,
                              utab, itab, utail, itail)

    def assemble(o):
        h0 = o[0].reshape(BATCH, _HD)
        h1 = o[1].reshape(BATCH, _HD)
        return jnp.concatenate([h0, h1], axis=1)

    return (assemble(uo), assemble(i1o), assemble(i2o))


def kernel(user_ids, item_ids_1, item_ids_2, user_emb, item_emb):
    return _run(user_ids, item_ids_1, item_ids_2, user_emb, item_emb)


# speedup vs baseline: 1.2461x; 1.0226x over previous
import functools

import jax
import jax.numpy as jnp
from jax import lax
from jax.experimental import pallas as pl
from jax.experimental.pallas import tpu as pltpu
from jax.experimental.pallas import tpu_sc as plsc

N_USERS = 100000
N_ITEMS = 1000000
EMB_DIM = 32
BATCH = 16384

_NC = 2
_NS = 16
_HD = EMB_DIM // _NC
_LPW = BATCH // _NS
_CHK = 128

_W = 1024
_WP = _W // 8
_NWIN_I = N_ITEMS // _W
_TAIL_I = N_ITEMS - _NWIN_I * _W
_NWIN_U = N_USERS // _W
_TAIL_U = N_USERS - _NWIN_U * _W
_JW_I = -(-_NWIN_I // _NS)
_JW_U = -(-_NWIN_U // _NS)
_SCR_I = _NWIN_I * _WP + _WP
_SCR_U = _NWIN_U * _WP + _WP


def _repack(win, pack, np_rows):
    lane = lax.iota(jnp.int32, 16)

    def rows(p2, carry):
        p = p2 * 2
        vals = []
        for dpp in range(2):
            for qg in range(8):
                qd = (lane + qg) & 7
                vals.append(plsc.load_gather(win, [lane, qd + 8 * (p + dpp)]))
        i = 0
        for dpp in range(2):
            prow = jnp.full((16,), 0, jnp.int32) + (p + dpp)
            for qg in range(8):
                qd = (lane + qg) & 7
                plsc.store_scatter(pack, [prow, (qd << 4) + lane], vals[i])
                i += 1
        return carry

    lax.fori_loop(0, np_rows // 2, rows, 0)


def _phase1(tab, scr, c, s, nwin, jw, win, pack, gsem, wsem):
    dbase = pl.multiple_of(c * _HD, _HD)

    def win_copy(j, buf, sem):
        w = s + _NS * j
        cb = pl.multiple_of(w * _W, _W)
        return pltpu.make_async_copy(
            tab.at[pl.ds(dbase, _HD), pl.ds(cb, _W)], buf, sem)

    def wb_copy(j, buf, sem):
        w = s + _NS * j
        rb = pl.multiple_of(w * _WP, _WP)
        return pltpu.make_async_copy(
            buf, scr.at[c, pl.ds(rb, _WP), :], sem)

    def start(j, buf, sem):
        @pl.when(s + _NS * j < nwin)
        def _():
            win_copy(j, buf, sem).start()

    def wait_g(j, buf, sem):
        @pl.when(s + _NS * j < nwin)
        def _():
            win_copy(j, buf, sem).wait()

    def start_wb(j, buf, sem):
        @pl.when(s + _NS * j < nwin)
        def _():
            wb_copy(j, buf, sem).start()

    def wait_wb(j, buf, sem):
        @pl.when(s + _NS * j < nwin)
        def _():
            wb_copy(j, buf, sem).wait()

    start(0, win[0], gsem[0])

    def body(p, carry):
        ja, jb = 2 * p, 2 * p + 1
        start(jb, win[1], gsem[1])

        @pl.when(p > 0)
        def _():
            wait_wb(ja - 2, pack[0], wsem[0])
        wait_g(ja, win[0], gsem[0])

        @pl.when(s + _NS * ja < nwin)
        def _():
            _repack(win[0], pack[0], _WP)
        start_wb(ja, pack[0], wsem[0])
        start(ja + 2, win[0], gsem[0])

        @pl.when(p > 0)
        def _():
            wait_wb(jb - 2, pack[1], wsem[1])
        wait_g(jb, win[1], gsem[1])

        @pl.when(s + _NS * jb < nwin)
        def _():
            _repack(win[1], pack[1], _WP)
        start_wb(jb, pack[1], wsem[1])
        return carry

    np2 = -(-jw // 2)
    lax.fori_loop(0, np2, body, 0)
    wait_wb(2 * np2 - 2, pack[0], wsem[0])
    wait_wb(2 * np2 - 1, pack[1], wsem[1])


def _tail(tail_ref, scr, c, s, row_base, win, pack, gsem, wsem):
    @pl.when(s == _NS - 1)
    def _():
        dbase = pl.multiple_of(c * _HD, _HD)
        cp = pltpu.make_async_copy(
            tail_ref.at[pl.ds(dbase, _HD), :], win, gsem)
        cp.start()
        cp.wait()
        _repack(win, pack, _WP)
        wb = pltpu.make_async_copy(
            pack, scr.at[c, pl.ds(row_base, _WP), :], wsem)
        wb.start()
        wb.wait()


def _sel_chunk(idx, kb, fetch, outb):
    lane = lax.iota(jnp.int32, 16)

    def sel(jg, carry):
        jloc = lane + jg * 16
        iv = idx[pl.ds(kb + jg * 16, 16)]
        colb = (iv & 7) << 4
        orow = jloc >> 3
        ocolb = (jloc & 7) << 4
        vals = []
        for dg in range(_HD):
            dd = (lane + dg) & (_HD - 1)
            vals.append(plsc.load_gather(fetch, [jloc, colb + dd]))
        for dg in range(_HD):
            dd = (lane + dg) & (_HD - 1)
            plsc.store_scatter(outb, [orow, ocolb + dd], vals[dg])
        return carry

    lax.fori_loop(0, _CHK // 16, sel, 0)


def _phase2_all(idxs, scrs, outs, c, s, fetch, blk, outb, gsem, wsem):
    nk = _LPW // _CHK
    tasks = [(idxs[t], scrs[t], outs[t], k)
             for t in range(3) for k in range(nk)]

    def prep_fire(ti, b):
        idx, scr, _, k = tasks[ti]
        kb = k * _CHK
        for t in range(_CHK // 16):
            blk[b][pl.ds(t * 16, 16)] = idx[pl.ds(kb + t * 16, 16)] >> 3
        pltpu.make_async_copy(scr.at[c].at[blk[b]], fetch[b], gsem[b]).start()

    prep_fire(0, 0)
    prep_fire(1, 1)
    for ti in range(len(tasks)):
        b = ti % 2
        idx, scr, out, k = tasks[ti]
        pltpu.make_async_copy(scr.at[c].at[blk[b]], fetch[b], gsem[b]).wait()
        if ti >= 2:
            _, _, pout, pk = tasks[ti - 2]
            pob = pl.multiple_of((s * _LPW + pk * _CHK) // 8, _CHK // 8)
            pltpu.make_async_copy(
                outb[b], pout.at[c, pl.ds(pob, _CHK // 8), :], wsem[b]).wait()
        _sel_chunk(idx, k * _CHK, fetch[b], outb[b])
        ob = pl.multiple_of((s * _LPW + k * _CHK) // 8, _CHK // 8)
        pltpu.make_async_copy(
            outb[b], out.at[c, pl.ds(ob, _CHK // 8), :], wsem[b]).start()
        if ti + 2 < len(tasks):
            prep_fire(ti + 2, b)
    for ti in (len(tasks) - 2, len(tasks) - 1):
        b = ti % 2
        _, _, out, k = tasks[ti]
        ob = pl.multiple_of((s * _LPW + k * _CHK) // 8, _CHK // 8)
        pltpu.make_async_copy(
            outb[b], out.at[c, pl.ds(ob, _CHK // 8), :], wsem[b]).wait()


def _kernel_body(uids, iids1, iids2, utab, itab, utail, itail,
                 uout, i1out, i2out, uscr, iscr,
                 idx_u, idx_1, idx_2, win0, win1, pack0, pack1,
                 fetch0, fetch1, blk0, blk1, outb0, outb1,
                 g0, g1, w0, w1, gf0, gf1, wf0, wf1):
    c = lax.axis_index("c")
    s = lax.axis_index("s")

    pltpu.sync_copy(uids.at[pl.ds(s * _LPW, _LPW)], idx_u)
    pltpu.sync_copy(iids1.at[pl.ds(s * _LPW, _LPW)], idx_1)
    pltpu.sync_copy(iids2.at[pl.ds(s * _LPW, _LPW)], idx_2)

    win = [win0, win1]
    pack = [pack0, pack1]
    gsem = [g0, g1]
    wsem = [w0, w1]

    _phase1(itab, iscr, c, s, _NWIN_I, _JW_I, win, pack, gsem, wsem)
    _phase1(utab, uscr, c, s, _NWIN_U, _JW_U, win, pack, gsem, wsem)
    _tail(itail, iscr, c, s, _NWIN_I * _WP, win0, pack0, g0, w0)
    _tail(utail, uscr, c, s, _NWIN_U * _WP, win0, pack0, g0, w0)

    plsc.subcore_barrier()

    _phase2_all((idx_u, idx_1, idx_2), (uscr, iscr, iscr),
                (uout, i1out, i2out), c, s,
                [fetch0, fetch1], [blk0, blk1], [outb0, outb1],
                [gf0, gf1], [wf0, wf1])


@jax.jit
def _run(user_ids, item_ids_1, item_ids_2, user_emb, item_emb):
    mesh = plsc.VectorSubcoreMesh(core_axis_name="c", subcore_axis_name="s")
    f32 = jnp.float32
    i32 = jnp.int32
    utab = user_emb.T
    itab = item_emb.T
    utail = jnp.pad(utab[:, _NWIN_U * _W:], ((0, 0), (0, _W - _TAIL_U)))
    itail = jnp.pad(itab[:, _NWIN_I * _W:], ((0, 0), (0, _W - _TAIL_I)))
    call = functools.partial(
        pl.kernel,
        mesh=mesh,
        compiler_params=pltpu.CompilerParams(needs_layout_passes=False),
        out_type=(
            jax.ShapeDtypeStruct((_NC, BATCH // 8, 128), f32),
            jax.ShapeDtypeStruct((_NC, BATCH // 8, 128), f32),
            jax.ShapeDtypeStruct((_NC, BATCH // 8, 128), f32),
            jax.ShapeDtypeStruct((_NC, _SCR_U, 128), f32),
            jax.ShapeDtypeStruct((_NC, _SCR_I, 128), f32),
        ),
        scratch_types=[
            pltpu.VMEM((_LPW,), i32),
            pltpu.VMEM((_LPW,), i32),
            pltpu.VMEM((_LPW,), i32),
            pltpu.VMEM((_HD, _W), f32),
            pltpu.VMEM((_HD, _W), f32),
            pltpu.VMEM((_WP, 128), f32),
            pltpu.VMEM((_WP, 128), f32),
            pltpu.VMEM((_CHK, 128), f32),
            pltpu.VMEM((_CHK, 128), f32),
            pltpu.VMEM((_CHK,), i32),
            pltpu.VMEM((_CHK,), i32),
            pltpu.VMEM((_CHK // 8, 128), f32),
            pltpu.VMEM((_CHK // 8, 128), f32),
            pltpu.SemaphoreType.DMA,
            pltpu.SemaphoreType.DMA,
            pltpu.SemaphoreType.DMA,
            pltpu.SemaphoreType.DMA,
            pltpu.SemaphoreType.DMA,
            pltpu.SemaphoreType.DMA,
            pltpu.SemaphoreType.DMA,
            pltpu.SemaphoreType.DMA,
        ],
    )(_kernel_body)
    uo, i1o, i2o, _, _ = call(user_ids.astype(i32), item_ids_1, item_ids_2,
                              utab, itab, utail, itail)

    def assemble(o):
        h0 = o[0].reshape(BATCH, _HD)
        h1 = o[1].reshape(BATCH, _HD)
        return jnp.concatenate([h0, h1], axis=1)

    return (assemble(uo), assemble(i1o), assemble(i2o))


def kernel(user_ids, item_ids_1, item_ids_2, user_emb, item_emb):
    return _run(user_ids, item_ids_1, item_ids_2, user_emb, item_emb)
